# baseline (speedup 1.0000x reference)
import functools

import jax
import jax.numpy as jnp
from jax import lax
from jax.experimental import pallas as pl
from jax.experimental.pallas import tpu as pltpu

N = 10000
E = 320000
B = 16
D_NODE = 128
D_EDGE = 16
D_GLOB = 32
D_EOUT = 64
D_NOUT = 128


def _h12_body(x_ref, w_ref, out_ref):
    out_ref[...] = jnp.dot(x_ref[...], w_ref[...],
                           preferred_element_type=jnp.float32)


def _node_precompute(x, Wa):
    blk = 2000
    return pl.pallas_call(
        _h12_body,
        grid=(N // blk,),
        in_specs=[
            pl.BlockSpec((blk, D_NODE), lambda i: (i, 0)),
            pl.BlockSpec((D_NODE, D_EOUT + D_NOUT), lambda i: (0, 0)),
        ],
        out_specs=pl.BlockSpec((blk, D_EOUT + D_NOUT), lambda i: (i, 0)),
        out_shape=jax.ShapeDtypeStruct((N, D_EOUT + D_NOUT), jnp.float32),
    )(x, Wa)


def _edge_body(g1_ref, ea_ref, wee_ref, be_ref, wne_ref, bn_ref, e_ref, q_ref):
    ea = jnp.dot(ea_ref[...], wee_ref[...], preferred_element_type=jnp.float32)
    e = jnp.maximum(g1_ref[...] + ea + be_ref[...], 0.0)
    e_ref[...] = e
    q_ref[...] = jnp.dot(e, wne_ref[...],
                         preferred_element_type=jnp.float32) + bn_ref[...]


def _edge_mlps(g1, edge_attr, We_e, be, Wn_e, bn):
    blk = 8000
    return pl.pallas_call(
        _edge_body,
        grid=(E // blk,),
        in_specs=[
            pl.BlockSpec((blk, D_EOUT), lambda i: (i, 0)),
            pl.BlockSpec((blk, D_EDGE), lambda i: (i, 0)),
            pl.BlockSpec((D_EDGE, D_EOUT), lambda i: (0, 0)),
            pl.BlockSpec((1, D_EOUT), lambda i: (0, 0)),
            pl.BlockSpec((D_EOUT, D_NOUT), lambda i: (0, 0)),
            pl.BlockSpec((1, D_NOUT), lambda i: (0, 0)),
        ],
        out_specs=[
            pl.BlockSpec((blk, D_EOUT), lambda i: (i, 0)),
            pl.BlockSpec((blk, D_NOUT), lambda i: (i, 0)),
        ],
        out_shape=[
            jax.ShapeDtypeStruct((E, D_EOUT), jnp.float32),
            jax.ShapeDtypeStruct((E, D_NOUT), jnp.float32),
        ],
    )(g1, edge_attr, We_e, be.reshape(1, -1), Wn_e, bn.reshape(1, -1))


def _node_glob_body(agg_ref, batch_ref, glob_ref, wn2a_ref, g2b_ref,
                    wgg_ref, wgm_ref, bg_ref, xnew_ref, unew_ref):
    agg = agg_ref[...]
    onehot = (batch_ref[...] == lax.broadcasted_iota(jnp.int32, (N, B), 1)
              ).astype(jnp.float32)
    sel = jnp.dot(onehot, g2b_ref[...], preferred_element_type=jnp.float32)
    x_new = jnp.maximum(
        jnp.dot(agg, wn2a_ref[...], preferred_element_type=jnp.float32) + sel,
        0.0)
    xnew_ref[...] = x_new
    cnt = jnp.maximum(jnp.sum(onehot, axis=0, keepdims=True), 1.0)
    ssum = lax.dot_general(onehot, x_new, (((0,), (0,)), ((), ())),
                           preferred_element_type=jnp.float32)
    mean = ssum / cnt.reshape(B, 1)
    u = (jnp.dot(glob_ref[...], wgg_ref[...], preferred_element_type=jnp.float32)
         + jnp.dot(mean, wgm_ref[...], preferred_element_type=jnp.float32)
         + bg_ref[...])
    unew_ref[...] = jnp.maximum(u, 0.0)


def _node_glob(agg, batch2d, glob, Wn2_a, g2b, Wg_g, Wg_m, bg):
    return pl.pallas_call(
        _node_glob_body,
        in_specs=[
            pl.BlockSpec((N, D_NOUT), lambda: (0, 0)),
            pl.BlockSpec((N, 1), lambda: (0, 0)),
            pl.BlockSpec((B, D_GLOB), lambda: (0, 0)),
            pl.BlockSpec((D_NOUT, D_NOUT), lambda: (0, 0)),
            pl.BlockSpec((B, D_NOUT), lambda: (0, 0)),
            pl.BlockSpec((D_GLOB, D_GOUT := 32), lambda: (0, 0)),
            pl.BlockSpec((D_NOUT, 32), lambda: (0, 0)),
            pl.BlockSpec((1, 32), lambda: (0, 0)),
        ],
        out_specs=[
            pl.BlockSpec((N, D_NOUT), lambda: (0, 0)),
            pl.BlockSpec((B, 32), lambda: (0, 0)),
        ],
        out_shape=[
            jax.ShapeDtypeStruct((N, D_NOUT), jnp.float32),
            jax.ShapeDtypeStruct((B, 32), jnp.float32),
        ],
    )(agg, batch2d, glob, Wn2_a, g2b, Wg_g, Wg_m, bg.reshape(1, -1))


def kernel(x, edge_index, edge_attr, glob, batch, We, be, Wn, bn, Wn2, bn2, Wg, bg):
    row = edge_index[0]
    col = edge_index[1]
    We_x, We_e = We[:D_NODE], We[D_NODE:]
    Wn_x, Wn_e = Wn[:D_NODE], Wn[D_NODE:]
    Wn2_a, Wn2_g = Wn2[:D_NOUT], Wn2[D_NOUT:]
    Wg_g, Wg_m = Wg[:D_GLOB], Wg[D_GLOB:]

    h12 = _node_precompute(x, jnp.concatenate([We_x, Wn_x], axis=1))
    h1 = h12[:, :D_EOUT]
    h2 = h12[:, D_EOUT:]

    g1 = jnp.take(h1, row, axis=0)

    e, q = _edge_mlps(g1, edge_attr, We_e, be, Wn_e, bn)

    m = q + jnp.take(h2, col, axis=0)
    agg = jax.ops.segment_max(m, row, num_segments=N)
    agg = jnp.where(jnp.isfinite(agg), agg, 0.0)

    g2b = glob @ Wn2_g + bn2
    x_new, u_new = _node_glob(agg, batch.reshape(N, 1), glob, Wn2_a, g2b,
                              Wg_g, Wg_m, bg)
    return (x_new, e, u_new)
